# Initial kernel scaffold; baseline (speedup 1.0000x reference)
import jax
import jax.numpy as jnp
from jax.experimental import pallas as pl


def kernel(atom_types, edge_types, n_per_mol, n_target, node_target, edge_target, valency_target):
    raise NotImplementedError("write your pallas kernel here")


# speedup vs baseline: 22.4252x; 22.4252x over previous
import functools

import jax
import jax.numpy as jnp
from jax import lax
from jax.experimental import pallas as pl
from jax.experimental.pallas import tpu as pltpu

_A = 16
_EB = 5
_VBINS = 190


def _tree_sum(xs):
    while len(xs) > 1:
        nxt = [xs[i] + xs[i + 1] for i in range(0, len(xs) - 1, 2)]
        if len(xs) % 2:
            nxt.append(xs[-1])
        xs = nxt
    return xs[0]


def _hist_kernel(n_ref, n3_ref, at_ref, ev_ref, tn_ref, ta_ref, te_ref,
                 tv_ref, out_ref, nh, ah, eh, vh, *, bb, n_atoms, n_grid):
    pid = pl.program_id(0)

    @pl.when(pid == 0)
    def _init():
        nh[...] = jnp.zeros_like(nh)
        ah[...] = jnp.zeros_like(ah)
        eh[...] = jnp.zeros_like(eh)
        vh[...] = jnp.zeros_like(vh)

    lane = lax.broadcasted_iota(jnp.int32, (1, 128), 1)
    idx2 = (lax.broadcasted_iota(jnp.int32, (2, 128), 0) * 128
            + lax.broadcasted_iota(jnp.int32, (2, 128), 1))

    n_blk = n_ref[...]
    n3 = n3_ref[...]
    at_blk = at_ref[...]
    ev_blk = ev_ref[...]

    pos = lax.broadcasted_iota(jnp.int32, (bb, n_atoms), 1)
    mask = pos < n_blk

    nv = jnp.clip(n_blk, 0, n_atoms)
    nh[...] += jnp.sum((nv == lane).astype(jnp.float32), axis=0,
                       keepdims=True)

    aw = jnp.where(mask, at_blk, 31)
    ah[...] += _tree_sum(
        [jnp.sum((aw == a).astype(jnp.float32)) *
         (lane == a).astype(jnp.float32) for a in range(_A)])

    pos_i = lax.broadcasted_iota(jnp.int32, (bb, n_atoms, n_atoms), 1)
    pos_j = lax.broadcasted_iota(jnp.int32, (bb, n_atoms, n_atoms), 2)
    mi = pos_i < n3
    mj = pos_j < n3
    pair = mi & mj
    emask = (pos_i < pos_j) & pair

    w = jnp.where(emask, ev_blk, 7)
    eh[...] += _tree_sum(
        [jnp.sum((w == e).astype(jnp.float32)) *
         (lane == e).astype(jnp.float32) for e in range(_EB)])

    etf = jnp.where(ev_blk == 4, jnp.float32(1.5),
                    ev_blk.astype(jnp.float32))
    val = jnp.sum(jnp.where(pair, etf, 0.0), axis=1)
    vi = jnp.where(mask,
                   jnp.clip(jnp.floor(val).astype(jnp.int32), 0, _VBINS - 1),
                   200)
    vh[...] += _tree_sum(
        [jnp.sum((vi == b).astype(jnp.float32)) *
         (idx2 == b).astype(jnp.float32) for b in range(_VBINS)])

    @pl.when(pid == n_grid - 1)
    def _fin():
        gen_n = nh[...] / jnp.sum(nh[...])
        gen_a = ah[...] / jnp.sum(ah[...])
        gen_e = eh[...] / jnp.sum(eh[...])
        gen_v = vh[...] / jnp.sum(vh[...])
        tn = tn_ref[...] / jnp.sum(tn_ref[...])
        ta = ta_ref[...] / jnp.sum(ta_ref[...])
        te = te_ref[...] / jnp.sum(te_ref[...])
        tv = tv_ref[...] / jnp.sum(tv_ref[...])
        n_mae = jnp.sum(jnp.abs(gen_n - tn)) / (n_atoms + 1)
        a_mae = jnp.sum(jnp.abs(gen_a - ta)) / _A
        e_mae = jnp.sum(jnp.abs(gen_e - te)) / _EB
        v_mae = jnp.sum(jnp.abs(gen_v - tv)) / _VBINS
        maes = (n_mae * (lane == 0) + a_mae * (lane == 1)
                + e_mae * (lane == 2) + v_mae * (lane == 3))
        out_ref[0:1, :] = gen_n
        out_ref[1:2, :] = gen_a
        out_ref[2:3, :] = gen_e
        out_ref[3:5, :] = gen_v
        out_ref[5:6, :] = maes
        out_ref[6:8, :] = jnp.zeros((2, 128), jnp.float32)


def kernel(atom_types, edge_types, n_per_mol, n_target, node_target,
           edge_target, valency_target):
    b, n = atom_types.shape
    bb = 256 if b % 256 == 0 else b
    n_grid = b // bb

    n2 = n_per_mol.reshape(b, 1)
    n3 = n_per_mol.reshape(b, 1, 1)
    tn = jnp.zeros((1, 128), jnp.float32).at[0, : n + 1].set(n_target)
    ta = jnp.zeros((1, 128), jnp.float32).at[0, :_A].set(node_target)
    te = jnp.zeros((1, 128), jnp.float32).at[0, :_EB].set(edge_target)
    tv = jnp.zeros((2, 128), jnp.float32).reshape(-1).at[:_VBINS].set(
        valency_target).reshape(2, 128)

    body = functools.partial(_hist_kernel, bb=bb, n_atoms=n, n_grid=n_grid)
    out = pl.pallas_call(
        body,
        grid=(n_grid,),
        in_specs=[
            pl.BlockSpec((bb, 1), lambda i: (i, 0)),
            pl.BlockSpec((bb, 1, 1), lambda i: (i, 0, 0)),
            pl.BlockSpec((bb, n), lambda i: (i, 0)),
            pl.BlockSpec((bb, n, n), lambda i: (i, 0, 0)),
            pl.BlockSpec((1, 128), lambda i: (0, 0)),
            pl.BlockSpec((1, 128), lambda i: (0, 0)),
            pl.BlockSpec((1, 128), lambda i: (0, 0)),
            pl.BlockSpec((2, 128), lambda i: (0, 0)),
        ],
        out_specs=pl.BlockSpec((8, 128), lambda i: (0, 0)),
        out_shape=jax.ShapeDtypeStruct((8, 128), jnp.float32),
        scratch_shapes=[
            pltpu.VMEM((1, 128), jnp.float32),
            pltpu.VMEM((1, 128), jnp.float32),
            pltpu.VMEM((1, 128), jnp.float32),
            pltpu.VMEM((2, 128), jnp.float32),
        ],
        compiler_params=pltpu.CompilerParams(
            dimension_semantics=("arbitrary",)),
    )(n2, n3, atom_types, edge_types, tn, ta, te, tv)

    return jnp.concatenate([
        out[0, : n + 1], out[1, :_A], out[2, :_EB],
        out[3, :], out[4, : _VBINS - 128], out[5, :4],
    ])


# speedup vs baseline: 42.7911x; 1.9082x over previous
import functools

import jax
import jax.numpy as jnp
from jax import lax
from jax.experimental import pallas as pl
from jax.experimental.pallas import tpu as pltpu

_A = 16
_EB = 5
_VBINS = 190


def _tree_sum(xs):
    while len(xs) > 1:
        nxt = [xs[i] + xs[i + 1] for i in range(0, len(xs) - 1, 2)]
        if len(xs) % 2:
            nxt.append(xs[-1])
        xs = nxt
    return xs[0]


def _hist_kernel(n_ref, n3_ref, at_ref, ev_ref, imap_ref, jmap_ref,
                 tn_ref, ta_ref, te_ref, tv_ref, out_ref,
                 nh, ah, eh, vh, *, bb, n_atoms, n_grid):
    pid = pl.program_id(0)

    @pl.when(pid == 0)
    def _init():
        nh[...] = jnp.zeros_like(nh)
        ah[...] = jnp.zeros_like(ah)
        eh[...] = jnp.zeros_like(eh)
        vh[...] = jnp.zeros_like(vh)

    lane = lax.broadcasted_iota(jnp.int32, (1, 128), 1)
    idx2 = (lax.broadcasted_iota(jnp.int32, (2, 128), 0) * 128
            + lax.broadcasted_iota(jnp.int32, (2, 128), 1))

    n_blk = n_ref[...]
    n3 = n3_ref[...]
    at_blk = at_ref[...]
    ev_blk = ev_ref[...]
    i_map = imap_ref[...]
    j_map = jmap_ref[...]

    pos = lax.broadcasted_iota(jnp.int32, (bb, n_atoms), 1)
    mask = pos < n_blk

    nv = jnp.clip(n_blk, 0, n_atoms)
    nh[...] += jnp.sum((nv == lane).astype(jnp.float32), axis=0,
                       keepdims=True)

    aw = jnp.where(mask, at_blk, 31)
    ah[...] += _tree_sum(
        [jnp.sum((aw == a).astype(jnp.float32)) *
         (lane == a).astype(jnp.float32) for a in range(_A)])

    mi = i_map < n3
    mj = j_map < n3
    emask = (i_map < j_map) & mj

    w = jnp.where(emask, ev_blk, 7)
    eh[...] += _tree_sum(
        [jnp.sum((w == e).astype(jnp.float32)) *
         (lane == e).astype(jnp.float32) for e in range(_EB)])

    etf = jnp.where(ev_blk == 4, jnp.float32(1.5),
                    ev_blk.astype(jnp.float32))
    s = jnp.sum(jnp.where(mi, etf, 0.0), axis=1)
    val = s[:, :n_atoms] + s[:, n_atoms:]
    vi = jnp.where(mask,
                   jnp.clip(jnp.floor(val).astype(jnp.int32), 0, _VBINS - 1),
                   200)
    vh[...] += _tree_sum(
        [jnp.sum((vi == b).astype(jnp.float32)) *
         (idx2 == b).astype(jnp.float32) for b in range(_VBINS)])

    @pl.when(pid == n_grid - 1)
    def _fin():
        gen_n = nh[...] / jnp.sum(nh[...])
        gen_a = ah[...] / jnp.sum(ah[...])
        gen_e = eh[...] / jnp.sum(eh[...])
        gen_v = vh[...] / jnp.sum(vh[...])
        tn = tn_ref[...] / jnp.sum(tn_ref[...])
        ta = ta_ref[...] / jnp.sum(ta_ref[...])
        te = te_ref[...] / jnp.sum(te_ref[...])
        tv = tv_ref[...] / jnp.sum(tv_ref[...])
        n_mae = jnp.sum(jnp.abs(gen_n - tn)) / (n_atoms + 1)
        a_mae = jnp.sum(jnp.abs(gen_a - ta)) / _A
        e_mae = jnp.sum(jnp.abs(gen_e - te)) / _EB
        v_mae = jnp.sum(jnp.abs(gen_v - tv)) / _VBINS
        maes = (n_mae * (lane == 0) + a_mae * (lane == 1)
                + e_mae * (lane == 2) + v_mae * (lane == 3))
        out_ref[0:1, :] = gen_n
        out_ref[1:2, :] = gen_a
        out_ref[2:3, :] = gen_e
        out_ref[3:5, :] = gen_v
        out_ref[5:6, :] = maes
        out_ref[6:8, :] = jnp.zeros((2, 128), jnp.float32)


def kernel(atom_types, edge_types, n_per_mol, n_target, node_target,
           edge_target, valency_target):
    b, n = atom_types.shape
    bb = 256 if b % 256 == 0 else b
    n_grid = b // bb

    n2 = n_per_mol.reshape(b, 1)
    n3 = n_per_mol.reshape(b, 1, 1)
    ev2 = edge_types.reshape(b, n // 2, 2 * n)
    rr = jnp.arange(n // 2, dtype=jnp.int32)[None, :, None]
    cc = jnp.arange(2 * n, dtype=jnp.int32)[None, None, :]
    i_map = jnp.broadcast_to(2 * rr + cc // n, (1, n // 2, 2 * n))
    j_map = jnp.broadcast_to(cc % n, (1, n // 2, 2 * n))
    tn = jnp.zeros((1, 128), jnp.float32).at[0, : n + 1].set(n_target)
    ta = jnp.zeros((1, 128), jnp.float32).at[0, :_A].set(node_target)
    te = jnp.zeros((1, 128), jnp.float32).at[0, :_EB].set(edge_target)
    tv = jnp.zeros((2, 128), jnp.float32).reshape(-1).at[:_VBINS].set(
        valency_target).reshape(2, 128)

    body = functools.partial(_hist_kernel, bb=bb, n_atoms=n, n_grid=n_grid)
    out = pl.pallas_call(
        body,
        grid=(n_grid,),
        in_specs=[
            pl.BlockSpec((bb, 1), lambda i: (i, 0)),
            pl.BlockSpec((bb, 1, 1), lambda i: (i, 0, 0)),
            pl.BlockSpec((bb, n), lambda i: (i, 0)),
            pl.BlockSpec((bb, n // 2, 2 * n), lambda i: (i, 0, 0)),
            pl.BlockSpec((1, n // 2, 2 * n), lambda i: (0, 0, 0)),
            pl.BlockSpec((1, n // 2, 2 * n), lambda i: (0, 0, 0)),
            pl.BlockSpec((1, 128), lambda i: (0, 0)),
            pl.BlockSpec((1, 128), lambda i: (0, 0)),
            pl.BlockSpec((1, 128), lambda i: (0, 0)),
            pl.BlockSpec((2, 128), lambda i: (0, 0)),
        ],
        out_specs=pl.BlockSpec((8, 128), lambda i: (0, 0)),
        out_shape=jax.ShapeDtypeStruct((8, 128), jnp.float32),
        scratch_shapes=[
            pltpu.VMEM((1, 128), jnp.float32),
            pltpu.VMEM((1, 128), jnp.float32),
            pltpu.VMEM((1, 128), jnp.float32),
            pltpu.VMEM((2, 128), jnp.float32),
        ],
        compiler_params=pltpu.CompilerParams(
            dimension_semantics=("arbitrary",)),
    )(n2, n3, atom_types, ev2, i_map, j_map, tn, ta, te, tv)

    return jnp.concatenate([
        out[0, : n + 1], out[1, :_A], out[2, :_EB],
        out[3, :], out[4, : _VBINS - 128], out[5, :4],
    ])
